# baseline (speedup 1.0000x reference)
import functools

import jax
import jax.numpy as jnp
import numpy as np
from jax import lax
from jax.experimental import pallas as pl
from jax.experimental.pallas import tpu as pltpu, tpu_sc as plsc

NC = 2
NS = 16
NW = NC * NS

CHUNK = 128


def _col_permutation(dim):
    perm = np.zeros(dim, dtype=np.int32)
    for d0 in range(0, dim, 32):
        for i in range(16):
            perm[d0 + 2 * i] = d0 + i
            perm[d0 + 2 * i + 1] = d0 + 16 + i
    return perm


def _make_sc_kernel(n_rows, chars, vocab, dim):
    rows_per_w = n_rows // NW
    n_chunks = rows_per_w // CHUNK
    assert n_chunks % 2 == 0
    idx_rows = (CHUNK * chars) // 128
    w_idx_rows = n_chunks * idx_rows

    mesh = plsc.VectorSubcoreMesh(core_axis_name="c", subcore_axis_name="s")

    @functools.partial(
        pl.kernel,
        mesh=mesh,
        compiler_params=pltpu.CompilerParams(
            use_tc_tiling_on_sc=False, needs_layout_passes=False),
        out_type=jax.ShapeDtypeStruct((n_rows, dim), jnp.float32),
        scratch_types=[
            pltpu.VMEM((w_idx_rows, 128), jnp.int32),
            pltpu.VMEM((2, CHUNK, dim), jnp.bfloat16),
            pltpu.VMEM((2, CHUNK, dim), jnp.float32),
            pltpu.SemaphoreType.DMA,
            pltpu.SemaphoreType.DMA,
            pltpu.SemaphoreType.DMA,
            pltpu.SemaphoreType.DMA,
            pltpu.SemaphoreType.DMA,
        ],
    )
    def embed_sum(idx_hbm, table_hbm, out_hbm, idx_v, acc_v, fout_v,
                  sem_i, sem_g0, sem_g1, sem_o0, sem_o1):
        wid = lax.axis_index("s") * NC + lax.axis_index("c")
        sem_g = [sem_g0, sem_g1]
        sem_o = [sem_o0, sem_o1]

        irow0 = pl.multiple_of(wid * w_idx_rows, 8)
        pltpu.sync_copy(idx_hbm.at[pl.ds(irow0, w_idx_rows)], idx_v)

        def base_of(g):
            return pl.multiple_of(wid * rows_per_w + g * CHUNK, CHUNK)

        def start_gathers(g, b):
            for j in range(idx_rows):
                pltpu.async_copy(
                    table_hbm.at[idx_v.at[g * idx_rows + j]],
                    acc_v.at[b],
                    sem_g[b],
                    add=True,
                )

        def wait_gathers(b):
            for _ in range(idx_rows):
                pltpu.make_async_copy(
                    table_hbm.at[idx_v.at[0]], acc_v.at[b], sem_g[b]).wait()

        def out_copy(g, b):
            return pltpu.make_async_copy(
                fout_v.at[b], out_hbm.at[pl.ds(base_of(g), CHUNK)], sem_o[b])

        zero = jnp.zeros((32,), jnp.bfloat16)

        def zero_acc(b):
            av = acc_v.at[b]

            def zb(c, carry):
                for d in range(dim // 32):
                    av[c, pl.ds(d * 32, 32)] = zero
                return carry

            lax.fori_loop(0, CHUNK, zb, 0, unroll=4)

        himask = jnp.full((16,), -65536, jnp.int32)

        def widen_acc(b):
            av = acc_v.at[b]
            fv = fout_v.at[b]

            def wb(c, carry):
                for d0 in range(0, dim, 32):
                    w = plsc.bitcast(av[c, pl.ds(d0, 32)], jnp.int32)
                    lo = plsc.bitcast(lax.shift_left(w, 16), jnp.float32)
                    hi = plsc.bitcast(lax.bitwise_and(w, himask), jnp.float32)
                    fv[c, pl.ds(d0, 16)] = lo
                    fv[c, pl.ds(d0 + 16, 16)] = hi
                return carry

            lax.fori_loop(0, CHUNK, wb, 0, unroll=2)

        def pair_body(gg, carry):
            for b in range(2):
                g = gg * 2 + b
                nb = 1 - b

                @pl.when(g >= 2)
                def _():
                    out_copy(g - 2, b).wait()

                zero_acc(b)
                start_gathers(g, b)

                @pl.when(g >= 1)
                def _():
                    wait_gathers(nb)
                    widen_acc(nb)
                    out_copy(g - 1, nb).start()

            return carry

        lax.fori_loop(0, n_chunks // 2, pair_body, 0)
        wait_gathers(1)
        widen_acc(1)
        out_copy(n_chunks - 1, 1).start()
        out_copy(n_chunks - 2, 0).wait()
        out_copy(n_chunks - 1, 1).wait()

    return embed_sum


def kernel(morphemes, table):
    b, s, chars = morphemes.shape
    vocab, dim = table.shape
    n_rows = b * s
    q = jnp.arange(n_rows * chars, dtype=jnp.int32)
    perm = (q // (128 * chars)) * (128 * chars) + (q % 128) * chars + (
        (q // 128) % chars)
    idx2d = jnp.take(morphemes.reshape(-1), perm).reshape(
        (n_rows * chars) // 128, 128)
    tbl = (
        table.astype(jnp.bfloat16)
        .reshape(vocab, dim // 32, 2, 16)
        .transpose(0, 1, 3, 2)
        .reshape(vocab, dim)
    )
    fn = _make_sc_kernel(n_rows, chars, vocab, dim)
    out = fn(idx2d, tbl)
    return out.reshape(b, s, dim)


# speedup vs baseline: 1.4729x; 1.4729x over previous
import functools

import jax
import jax.numpy as jnp
import numpy as np
from jax import lax
from jax.experimental import pallas as pl
from jax.experimental.pallas import tpu as pltpu, tpu_sc as plsc

NC = 2
NS = 16
NW = NC * NS

CHUNK = 128


def _col_permutation(dim):
    perm = np.zeros(dim, dtype=np.int32)
    for d0 in range(0, dim, 32):
        for i in range(16):
            perm[d0 + 2 * i] = d0 + i
            perm[d0 + 2 * i + 1] = d0 + 16 + i
    return perm


def _make_sc_kernel(n_rows, chars, vocab, dim):
    rows_per_w = n_rows // NW
    n_chunks = rows_per_w // CHUNK
    assert n_chunks % 2 == 0
    idx_rows = (CHUNK * chars) // 128
    w_idx_rows = n_chunks * idx_rows

    mesh = plsc.VectorSubcoreMesh(core_axis_name="c", subcore_axis_name="s")

    @functools.partial(
        pl.kernel,
        mesh=mesh,
        compiler_params=pltpu.CompilerParams(
            use_tc_tiling_on_sc=False, needs_layout_passes=False),
        out_type=jax.ShapeDtypeStruct((n_rows, dim), jnp.float32),
        scratch_types=[
            pltpu.VMEM((w_idx_rows, 128), jnp.int32),
            pltpu.VMEM((2, CHUNK, dim), jnp.bfloat16),
            pltpu.VMEM((2, CHUNK, dim), jnp.float32),
            pltpu.SemaphoreType.DMA,
            pltpu.SemaphoreType.DMA,
            pltpu.SemaphoreType.DMA,
            pltpu.SemaphoreType.DMA,
            pltpu.SemaphoreType.DMA,
        ],
    )
    def embed_sum(idx_hbm, table_hbm, out_hbm, idx_v, acc_v, fout_v,
                  sem_i, sem_g0, sem_g1, sem_o0, sem_o1):
        wid = lax.axis_index("s") * NC + lax.axis_index("c")
        sem_g = [sem_g0, sem_g1]
        sem_o = [sem_o0, sem_o1]

        irow0 = pl.multiple_of(wid * w_idx_rows, 8)
        pltpu.sync_copy(idx_hbm.at[pl.ds(irow0, w_idx_rows)], idx_v)

        def base_of(g):
            return pl.multiple_of(wid * rows_per_w + g * CHUNK, CHUNK)

        def start_gathers(g, b):
            for j in range(idx_rows):
                pltpu.async_copy(
                    table_hbm.at[idx_v.at[g * idx_rows + j]],
                    acc_v.at[b],
                    sem_g[b],
                    add=True,
                )

        def wait_gathers(b):
            for _ in range(idx_rows):
                pltpu.make_async_copy(
                    table_hbm.at[idx_v.at[0]], acc_v.at[b], sem_g[b]).wait()

        def out_copy(g, b):
            return pltpu.make_async_copy(
                fout_v.at[b], out_hbm.at[pl.ds(base_of(g), CHUNK)], sem_o[b])

        zero = jnp.zeros((32,), jnp.bfloat16)

        def zero_acc(b):
            av = acc_v.at[b]

            def zb(c, carry):
                for d in range(dim // 32):
                    av[c, pl.ds(d * 32, 32)] = zero
                return carry

            lax.fori_loop(0, CHUNK, zb, 0, unroll=4)

        himask = jnp.full((16,), -65536, jnp.int32)

        def widen_acc(b):
            av = acc_v.at[b]
            fv = fout_v.at[b]

            def wb(c, carry):
                for d0 in range(0, dim, 32):
                    w = plsc.bitcast(av[c, pl.ds(d0, 32)], jnp.int32)
                    lo = plsc.bitcast(lax.shift_left(w, 16), jnp.float32)
                    hi = plsc.bitcast(lax.bitwise_and(w, himask), jnp.float32)
                    fv[c, pl.ds(d0, 16)] = lo
                    fv[c, pl.ds(d0 + 16, 16)] = hi
                return carry

            lax.fori_loop(0, CHUNK, wb, 0, unroll=2)

        def pair_body(gg, carry):
            for b in range(2):
                g = gg * 2 + b
                nb = 1 - b

                @pl.when(g >= 2)
                def _():
                    out_copy(g - 2, b).wait()

                zero_acc(b)
                start_gathers(g, b)

                @pl.when(g >= 1)
                def _():
                    wait_gathers(nb)
                    widen_acc(nb)
                    out_copy(g - 1, nb).start()

            return carry

        lax.fori_loop(0, n_chunks // 2, pair_body, 0)
        wait_gathers(1)
        widen_acc(1)
        out_copy(n_chunks - 1, 1).start()
        out_copy(n_chunks - 2, 0).wait()
        out_copy(n_chunks - 1, 1).wait()

    return embed_sum


def kernel(morphemes, table):
    b, s, chars = morphemes.shape
    vocab, dim = table.shape
    n_rows = b * s
    idx2d = (
        morphemes.reshape(n_rows // 128, 128, chars)
        .transpose(0, 2, 1)
        .reshape((n_rows * chars) // 128, 128)
    )
    tbl = (
        table.astype(jnp.bfloat16)
        .reshape(vocab, dim // 32, 2, 16)
        .transpose(0, 1, 3, 2)
        .reshape(vocab, dim)
    )
    fn = _make_sc_kernel(n_rows, chars, vocab, dim)
    out = fn(idx2d, tbl)
    return out.reshape(b, s, dim)


# speedup vs baseline: 1.5565x; 1.0568x over previous
import functools

import jax
import jax.numpy as jnp
import numpy as np
from jax import lax
from jax.experimental import pallas as pl
from jax.experimental.pallas import tpu as pltpu, tpu_sc as plsc

NC = 2
NS = 16
NW = NC * NS

CHUNK = 128


def _col_permutation(dim):
    perm = np.zeros(dim, dtype=np.int32)
    for d0 in range(0, dim, 32):
        for i in range(16):
            perm[d0 + 2 * i] = d0 + i
            perm[d0 + 2 * i + 1] = d0 + 16 + i
    return perm


def _make_sc_kernel(n_rows, chars, vocab, dim):
    rows_per_w = n_rows // NW
    n_chunks = rows_per_w // CHUNK
    assert n_chunks % 2 == 0
    idx_rows = (CHUNK * chars) // 128
    w_idx_rows = n_chunks * idx_rows

    mesh = plsc.VectorSubcoreMesh(core_axis_name="c", subcore_axis_name="s")

    @functools.partial(
        pl.kernel,
        mesh=mesh,
        compiler_params=pltpu.CompilerParams(
            use_tc_tiling_on_sc=False, needs_layout_passes=False),
        out_type=jax.ShapeDtypeStruct((n_rows, dim), jnp.float32),
        scratch_types=[
            pltpu.VMEM((w_idx_rows, 128), jnp.int32),
            pltpu.VMEM((2, CHUNK, dim), jnp.bfloat16),
            pltpu.VMEM((2, CHUNK, dim), jnp.float32),
            pltpu.SemaphoreType.DMA,
            pltpu.SemaphoreType.DMA,
            pltpu.SemaphoreType.DMA,
            pltpu.SemaphoreType.DMA,
            pltpu.SemaphoreType.DMA,
        ],
    )
    def embed_sum(idx_hbm, table_hbm, out_hbm, idx_v, acc_v, fout_v,
                  sem_i, sem_g0, sem_g1, sem_o0, sem_o1):
        wid = lax.axis_index("s") * NC + lax.axis_index("c")
        sem_g = [sem_g0, sem_g1]
        sem_o = [sem_o0, sem_o1]

        irow0 = pl.multiple_of(wid * w_idx_rows, 8)
        pltpu.sync_copy(idx_hbm.at[pl.ds(irow0, w_idx_rows)], idx_v)

        def base_of(g):
            return pl.multiple_of(wid * rows_per_w + g * CHUNK, CHUNK)

        def start_gathers(g, b):
            for j in range(idx_rows):
                pltpu.async_copy(
                    table_hbm.at[idx_v.at[g * idx_rows + j]],
                    acc_v.at[b],
                    sem_g[b],
                    add=True,
                )

        def wait_gathers(b):
            for _ in range(idx_rows):
                pltpu.make_async_copy(
                    table_hbm.at[idx_v.at[0]], acc_v.at[b], sem_g[b]).wait()

        def out_copy(g, b):
            return pltpu.make_async_copy(
                fout_v.at[b], out_hbm.at[pl.ds(base_of(g), CHUNK)], sem_o[b])

        zero = jnp.zeros((32,), jnp.bfloat16)

        def zero_acc(b):
            av = acc_v.at[b]

            def zb(c, carry):
                for d in range(dim // 32):
                    av[c, pl.ds(d * 32, 32)] = zero
                return carry

            lax.fori_loop(0, CHUNK, zb, 0, unroll=4)

        himask = jnp.full((16,), -65536, jnp.int32)

        def widen_acc(b):
            av = acc_v.at[b]
            fv = fout_v.at[b]

            def wb(c, carry):
                for d0 in range(0, dim, 32):
                    w = plsc.bitcast(av[c, pl.ds(d0, 32)], jnp.int32)
                    lo = plsc.bitcast(lax.shift_left(w, 16), jnp.float32)
                    hi = plsc.bitcast(lax.bitwise_and(w, himask), jnp.float32)
                    fv[c, pl.ds(d0, 16)] = lo
                    fv[c, pl.ds(d0 + 16, 16)] = hi
                return carry

            lax.fori_loop(0, CHUNK, wb, 0, unroll=2)

        def pair_body(gg, carry):
            for b in range(2):
                g = gg * 2 + b
                nb = 1 - b

                @pl.when(g >= 2)
                def _():
                    out_copy(g - 2, b).wait()

                zero_acc(b)
                start_gathers(g, b)

                @pl.when(g >= 1)
                def _():
                    wait_gathers(nb)
                    widen_acc(nb)
                    out_copy(g - 1, nb).start()

            return carry

        lax.fori_loop(0, n_chunks // 2, pair_body, 0)
        wait_gathers(1)
        widen_acc(1)
        out_copy(n_chunks - 1, 1).start()
        out_copy(n_chunks - 2, 0).wait()
        out_copy(n_chunks - 1, 1).wait()

    return embed_sum


def kernel(morphemes, table):
    b, s, chars = morphemes.shape
    vocab, dim = table.shape
    n_rows = b * s
    idx2d = (
        morphemes.reshape(n_rows // 128, 128, chars)
        .transpose(0, 2, 1)
        .reshape((n_rows * chars) // 128, 128)
    )
    tbl = table.astype(jnp.bfloat16)[:, _col_permutation(dim)]
    fn = _make_sc_kernel(n_rows, chars, vocab, dim)
    out = fn(idx2d, tbl)
    return out.reshape(b, s, dim)
